# baseline (speedup 1.0000x reference)
import functools

import jax
import jax.numpy as jnp
from jax import lax
from jax.experimental import pallas as pl
from jax.experimental.pallas import tpu as pltpu
from jax.experimental.pallas import tpu_sc as plsc

N_NODES = 10000
N_EDGES = 320000
F = 128
L = 16
NC = 2
NS = 16
NW = NC * NS
B = 128
EPT = N_EDGES // NW
NFULL = EPT // B
REM = EPT - NFULL * B
RPT = 624
RPT_LAST = N_NODES - (NS - 1) * RPT
BJORCK_ITER = 10
BJORCK_BETA = 0.5


def _tc_body(x_ref, w_ref, out_ref):
    prec = lax.Precision.HIGHEST

    def mm(a, b, dims):
        return lax.dot_general(a, b, (dims, ((), ())), precision=prec,
                               preferred_element_type=jnp.float32)

    v = w_ref[...] * (1.0 / 128.0)

    def it(_, v):
        vvt = mm(v, v, ((1,), (1,)))
        return (1.0 + BJORCK_BETA) * v - BJORCK_BETA * mm(vvt, v, ((1,), (0,)))

    v = lax.fori_loop(0, BJORCK_ITER, it, v)
    out_ref[...] = mm(x_ref[...], v, ((1,), (0,)))


_tc_support = pl.pallas_call(
    _tc_body,
    out_shape=jax.ShapeDtypeStruct((N_NODES, F), jnp.float32),
)


def _tc_add_body(a_ref, b_ref, out_ref):
    out_ref[...] = a_ref[...] + b_ref[...]


_tc_add = pl.pallas_call(
    _tc_add_body,
    out_shape=jax.ShapeDtypeStruct((N_NODES, F), jnp.float32),
)


_sc_mesh = plsc.VectorSubcoreMesh(core_axis_name="c", subcore_axis_name="s")


_UNROLL = 6
_NSUPER = NFULL // _UNROLL
assert _NSUPER * _UNROLL == NFULL

_DNUMS = lax.GatherDimensionNumbers(
    offset_dims=(), collapsed_slice_dims=(0,), start_index_map=(0,))


@functools.partial(
    pl.kernel,
    out_type=(jax.ShapeDtypeStruct((N_NODES, F), jnp.float32),
              jax.ShapeDtypeStruct((N_NODES, F), jnp.float32)),
    mesh=_sc_mesh,
    scratch_types=[
        pltpu.VMEM((3, B), jnp.int32),
        pltpu.VMEM((3, B), jnp.int32),
        pltpu.VMEM((3, B), jnp.float32),
        pltpu.VMEM((2, B, F), jnp.float32),
        pltpu.VMEM((REM,), jnp.int32),
        pltpu.VMEM((REM,), jnp.int32),
        pltpu.VMEM((REM,), jnp.float32),
        pltpu.VMEM((REM, F), jnp.float32),
        pltpu.VMEM_SHARED((N_NODES, F), jnp.float32),
        pltpu.SemaphoreType.DMA,
        pltpu.SemaphoreType.DMA,
        pltpu.SemaphoreType.DMA,
        pltpu.SemaphoreType.DMA,
        pltpu.SemaphoreType.DMA,
        pltpu.SemaphoreType.DMA,
        pltpu.SemaphoreType.DMA,
        pltpu.SemaphoreType.DMA,
    ],
)
def _sc_spmm(sup_ref, row_ref, col_ref, wt_ref, out0_ref, out1_ref,
             rowi_v, coli_v, wtc_v, rows_v,
             rowr_v, colr_v, wtr_v, rowsr_v,
             accum, asem0, asem1, asem2, gsem0, gsem1, ssem0, ssem1, sem):
    cid = lax.axis_index("c")
    sid = lax.axis_index("s")
    rbase = sid * RPT
    ebase = (cid * NS + sid) * EPT
    asems = (asem0, asem1, asem2)
    gsems = (gsem0, gsem1)
    ssems = (ssem0, ssem1)

    def _issue_a(j3, k):
        off = ebase + k * B
        pltpu.async_copy(row_ref.at[pl.ds(off, B)], rowi_v.at[j3], asems[j3])
        pltpu.async_copy(col_ref.at[pl.ds(off, B)], coli_v.at[j3], asems[j3])
        pltpu.async_copy(wt_ref.at[pl.ds(off, B)], wtc_v.at[j3], asems[j3])

    def _wait_a(j3):
        pltpu.make_async_copy(row_ref.at[pl.ds(0, B)], rowi_v.at[j3],
                              asems[j3]).wait()
        pltpu.make_async_copy(col_ref.at[pl.ds(0, B)], coli_v.at[j3],
                              asems[j3]).wait()
        pltpu.make_async_copy(wt_ref.at[pl.ds(0, B)], wtc_v.at[j3],
                              asems[j3]).wait()

    def _issue_b(b2, j3):
        pltpu.async_copy(sup_ref.at[coli_v.at[j3]], rows_v.at[b2], gsems[b2])

    def _wait_b(b2, j3):
        pltpu.make_async_copy(sup_ref.at[coli_v.at[j3]], rows_v.at[b2],
                              gsems[b2]).wait()

    def _issue_s(b2, j3):
        pltpu.async_copy(rows_v.at[b2], accum.at[rowi_v.at[j3]], ssems[b2],
                         add=True)

    def _wait_s(b2, j3):
        pltpu.make_async_copy(rows_v.at[b2], accum.at[rowi_v.at[j3]],
                              ssems[b2]).wait()

    def _multiply(rows, wtc, n):
        def _grp(g, c):
            w = wtc[pl.ds(g * L, L)]

            def _mul_body(i, c2):
                for u in range(4):
                    ii = i * 4 + u
                    w16 = w
                    e = g * L + ii
                    for j in range(F // L):
                        sl = pl.ds(j * L, L)
                        rows[e, sl] = rows[e, sl] * w16
                return c2

            return lax.fori_loop(0, L // 4, _mul_body, c)

        lax.fori_loop(0, n // L, _grp, 0)

    def _zero_rows(e, c):
        for j in range(F // L):
            rowsr_v[e, pl.ds(j * L, L)] = jnp.zeros((L,), jnp.float32)
        return c

    lax.fori_loop(0, REM, _zero_rows, 0)

    def _zero_range(start, n):
        for o in range(0, (n // REM) * REM, REM):
            pltpu.async_copy(rowsr_v, accum.at[pl.ds(start + o, REM)], sem)
        tail = n - (n // REM) * REM
        if tail:
            pltpu.async_copy(rowsr_v.at[pl.ds(0, tail)],
                             accum.at[pl.ds(start + (n // REM) * REM, tail)],
                             sem)

    def _zero_drain(n):
        for o in range(0, (n // REM) * REM, REM):
            pltpu.make_async_copy(rowsr_v, accum.at[pl.ds(rbase, REM)],
                                  sem).wait()
        tail = n - (n // REM) * REM
        if tail:
            pltpu.make_async_copy(rowsr_v.at[pl.ds(0, tail)],
                                  accum.at[pl.ds(rbase, tail)], sem).wait()

    @pl.when(sid < NS - 1)
    def _():
        _zero_range(rbase, RPT)

    @pl.when(sid == NS - 1)
    def _():
        _zero_range(rbase, RPT_LAST)

    _issue_a(0, 0)
    _issue_a(1, 1)

    @pl.when(sid < NS - 1)
    def _():
        _zero_drain(RPT)

    @pl.when(sid == NS - 1)
    def _():
        _zero_drain(RPT_LAST)

    plsc.subcore_barrier()

    _wait_a(0)
    _issue_b(0, 0)

    def _super(sup, c):
        k0 = sup * _UNROLL
        for j in range(_UNROLL):
            kk = k0 + j
            b2, j3 = j % 2, j % 3
            _wait_b(b2, j3)

            @pl.when(kk >= 1)
            def _():
                _wait_s((j + 1) % 2, (j + 2) % 3)

            @pl.when(kk + 2 < NFULL)
            def _():
                _issue_a((j + 2) % 3, kk + 2)

            @pl.when(kk + 1 < NFULL)
            def _():
                _wait_a((j + 1) % 3)
                _issue_b((j + 1) % 2, (j + 1) % 3)

            _multiply(rows_v.at[b2], wtc_v.at[j3], B)
            _issue_s(b2, j3)
        return c

    lax.fori_loop(0, _NSUPER, _super, 0)
    _wait_s((NFULL - 1) % 2, (NFULL - 1) % 3)

    if REM:
        off = ebase + NFULL * B
        pltpu.sync_copy(row_ref.at[pl.ds(off, REM)], rowr_v)
        pltpu.sync_copy(col_ref.at[pl.ds(off, REM)], colr_v)
        pltpu.sync_copy(wt_ref.at[pl.ds(off, REM)], wtr_v)
        pltpu.async_copy(sup_ref.at[colr_v], rowsr_v, sem).wait()
        _multiply(rowsr_v, wtr_v, REM)
        pltpu.sync_copy(rowsr_v, accum.at[rowr_v], add=True)
    plsc.subcore_barrier()

    def _writeout(out_ref, n):
        for o in range(0, (n // B) * B, B):
            pltpu.async_copy(accum.at[pl.ds(rbase + o, B)],
                             out_ref.at[pl.ds(rbase + o, B)], sem)
        tail = n - (n // B) * B
        if tail:
            pltpu.async_copy(accum.at[pl.ds(rbase + (n // B) * B, tail)],
                             out_ref.at[pl.ds(rbase + (n // B) * B, tail)],
                             sem)
        for o in range(0, (n // B) * B, B):
            pltpu.make_async_copy(accum.at[pl.ds(rbase, B)],
                                  out_ref.at[pl.ds(rbase, B)], sem).wait()
        if tail:
            pltpu.make_async_copy(accum.at[pl.ds(rbase, tail)],
                                  out_ref.at[pl.ds(rbase, tail)], sem).wait()

    for c, out_ref in ((0, out0_ref), (1, out1_ref)):
        @pl.when((cid == c) & (sid < NS - 1))
        def _(out_ref=out_ref):
            _writeout(out_ref, RPT)

        @pl.when((cid == c) & (sid == NS - 1))
        def _(out_ref=out_ref):
            _writeout(out_ref, RPT_LAST)


def kernel(input, edge_index, edge_weight, W):
    support = _tc_support(input, W)
    row = edge_index[0].astype(jnp.int32)
    col = edge_index[1].astype(jnp.int32)
    o0, o1 = _sc_spmm(support, row, col, edge_weight.astype(jnp.float32))
    return _tc_add(o0, o1)


# speedup vs baseline: 2.6627x; 2.6627x over previous
import functools

import jax
import jax.numpy as jnp
from jax import lax
from jax.experimental import pallas as pl
from jax.experimental.pallas import tpu as pltpu
from jax.experimental.pallas import tpu_sc as plsc

N_NODES = 10000
N_EDGES = 320000
F = 128
L = 16
NC = 2
NS = 16
NW = NC * NS
B = 128
EPT = N_EDGES // NW
NFULL = EPT // B
REM = EPT - NFULL * B
RPT = 624
RPT_LAST = N_NODES - (NS - 1) * RPT
BJORCK_ITER = 10
BJORCK_BETA = 0.5


def _tc_body(x_ref, w_ref, out_ref):
    prec = lax.Precision.HIGHEST

    def mm(a, b, dims):
        return lax.dot_general(a, b, (dims, ((), ())), precision=prec,
                               preferred_element_type=jnp.float32)

    v = w_ref[...] * (1.0 / 128.0)

    def it(_, v):
        vvt = mm(v, v, ((1,), (1,)))
        return (1.0 + BJORCK_BETA) * v - BJORCK_BETA * mm(vvt, v, ((1,), (0,)))

    v = lax.fori_loop(0, BJORCK_ITER, it, v)
    out_ref[...] = mm(x_ref[...], v, ((1,), (0,)))


_tc_support = pl.pallas_call(
    _tc_body,
    out_shape=jax.ShapeDtypeStruct((N_NODES, F), jnp.float32),
)


def _tc_add_body(a_ref, b_ref, out_ref):
    out_ref[...] = a_ref[...] + b_ref[...]


_tc_add = pl.pallas_call(
    _tc_add_body,
    out_shape=jax.ShapeDtypeStruct((N_NODES, F), jnp.float32),
)


_sc_mesh = plsc.VectorSubcoreMesh(core_axis_name="c", subcore_axis_name="s")


_UNROLL = 6
_NSUPER = NFULL // _UNROLL
assert _NSUPER * _UNROLL == NFULL

_DNUMS = lax.GatherDimensionNumbers(
    offset_dims=(), collapsed_slice_dims=(0,), start_index_map=(0,))


@functools.partial(
    pl.kernel,
    out_type=(jax.ShapeDtypeStruct((N_NODES, F), jnp.float32),
              jax.ShapeDtypeStruct((N_NODES, F), jnp.float32)),
    mesh=_sc_mesh,
    scratch_types=[
        pltpu.VMEM((3, B), jnp.int32),
        pltpu.VMEM((3, B), jnp.int32),
        pltpu.VMEM((3, B), jnp.float32),
        pltpu.VMEM((2, B, F), jnp.float32),
        pltpu.VMEM((REM,), jnp.int32),
        pltpu.VMEM((REM,), jnp.int32),
        pltpu.VMEM((REM,), jnp.float32),
        pltpu.VMEM((REM, F), jnp.float32),
        pltpu.VMEM_SHARED((N_NODES, F), jnp.float32),
        pltpu.SemaphoreType.DMA,
        pltpu.SemaphoreType.DMA,
        pltpu.SemaphoreType.DMA,
        pltpu.SemaphoreType.DMA,
        pltpu.SemaphoreType.DMA,
        pltpu.SemaphoreType.DMA,
        pltpu.SemaphoreType.DMA,
        pltpu.SemaphoreType.DMA,
    ],
)
def _sc_spmm(sup_ref, row_ref, col_ref, wt_ref, out0_ref, out1_ref,
             rowi_v, coli_v, wtc_v, rows_v,
             rowr_v, colr_v, wtr_v, rowsr_v,
             accum, asem0, asem1, asem2, gsem0, gsem1, ssem0, ssem1, sem):
    cid = lax.axis_index("c")
    sid = lax.axis_index("s")
    rbase = sid * RPT
    ebase = (cid * NS + sid) * EPT
    asems = (asem0, asem1, asem2)
    gsems = (gsem0, gsem1)
    ssems = (ssem0, ssem1)

    def _issue_a(j3, k):
        off = ebase + k * B
        pltpu.async_copy(row_ref.at[pl.ds(off, B)], rowi_v.at[j3], asems[j3])
        pltpu.async_copy(col_ref.at[pl.ds(off, B)], coli_v.at[j3], asems[j3])
        pltpu.async_copy(wt_ref.at[pl.ds(off, B)], wtc_v.at[j3], asems[j3])

    def _wait_a(j3):
        pltpu.make_async_copy(row_ref.at[pl.ds(0, B)], rowi_v.at[j3],
                              asems[j3]).wait()
        pltpu.make_async_copy(col_ref.at[pl.ds(0, B)], coli_v.at[j3],
                              asems[j3]).wait()
        pltpu.make_async_copy(wt_ref.at[pl.ds(0, B)], wtc_v.at[j3],
                              asems[j3]).wait()

    def _issue_b(b2, j3):
        pltpu.async_copy(sup_ref.at[coli_v.at[j3]], rows_v.at[b2], gsems[b2])

    def _wait_b(b2, j3):
        pltpu.make_async_copy(sup_ref.at[coli_v.at[j3]], rows_v.at[b2],
                              gsems[b2]).wait()

    def _issue_s(b2, j3):
        pltpu.async_copy(rows_v.at[b2], accum.at[rowi_v.at[j3]], ssems[b2],
                         add=True)

    def _wait_s(b2, j3):
        pltpu.make_async_copy(rows_v.at[b2], accum.at[rowi_v.at[j3]],
                              ssems[b2]).wait()

    def _multiply(rows, wtc, n):
        @plsc.parallel_loop(0, n, 1, unroll=4)
        def _(e):
            g16 = (e // L) * L
            w = wtc[pl.ds(g16, L)]
            w16 = lax.gather(
                w, lax.broadcast(e - g16, (L,))[:, None], _DNUMS,
                slice_sizes=(1,),
                mode=lax.GatherScatterMode.PROMISE_IN_BOUNDS)
            for j in range(F // L):
                sl = pl.ds(j * L, L)
                rows[e, sl] = rows[e, sl] * w16

    def _zero_rows(e, c):
        for j in range(F // L):
            rowsr_v[e, pl.ds(j * L, L)] = jnp.zeros((L,), jnp.float32)
        return c

    lax.fori_loop(0, REM, _zero_rows, 0)

    def _zero_range(start, n):
        for o in range(0, (n // REM) * REM, REM):
            pltpu.async_copy(rowsr_v, accum.at[pl.ds(start + o, REM)], sem)
        tail = n - (n // REM) * REM
        if tail:
            pltpu.async_copy(rowsr_v.at[pl.ds(0, tail)],
                             accum.at[pl.ds(start + (n // REM) * REM, tail)],
                             sem)

    def _zero_drain(n):
        for o in range(0, (n // REM) * REM, REM):
            pltpu.make_async_copy(rowsr_v, accum.at[pl.ds(rbase, REM)],
                                  sem).wait()
        tail = n - (n // REM) * REM
        if tail:
            pltpu.make_async_copy(rowsr_v.at[pl.ds(0, tail)],
                                  accum.at[pl.ds(rbase, tail)], sem).wait()

    @pl.when(sid < NS - 1)
    def _():
        _zero_range(rbase, RPT)

    @pl.when(sid == NS - 1)
    def _():
        _zero_range(rbase, RPT_LAST)

    _issue_a(0, 0)
    _issue_a(1, 1)

    @pl.when(sid < NS - 1)
    def _():
        _zero_drain(RPT)

    @pl.when(sid == NS - 1)
    def _():
        _zero_drain(RPT_LAST)

    plsc.subcore_barrier()

    _wait_a(0)
    _issue_b(0, 0)

    def _super(sup, c):
        k0 = sup * _UNROLL
        for j in range(_UNROLL):
            kk = k0 + j
            b2, j3 = j % 2, j % 3
            _wait_b(b2, j3)

            @pl.when(kk >= 1)
            def _():
                _wait_s((j + 1) % 2, (j + 2) % 3)

            @pl.when(kk + 2 < NFULL)
            def _():
                _issue_a((j + 2) % 3, kk + 2)

            @pl.when(kk + 1 < NFULL)
            def _():
                _wait_a((j + 1) % 3)
                _issue_b((j + 1) % 2, (j + 1) % 3)

            _multiply(rows_v.at[b2], wtc_v.at[j3], B)
            _issue_s(b2, j3)
        return c

    lax.fori_loop(0, _NSUPER, _super, 0)
    _wait_s((NFULL - 1) % 2, (NFULL - 1) % 3)

    if REM:
        off = ebase + NFULL * B
        pltpu.sync_copy(row_ref.at[pl.ds(off, REM)], rowr_v)
        pltpu.sync_copy(col_ref.at[pl.ds(off, REM)], colr_v)
        pltpu.sync_copy(wt_ref.at[pl.ds(off, REM)], wtr_v)
        pltpu.async_copy(sup_ref.at[colr_v], rowsr_v, sem).wait()
        _multiply(rowsr_v, wtr_v, REM)
        pltpu.sync_copy(rowsr_v, accum.at[rowr_v], add=True)
    plsc.subcore_barrier()

    def _writeout(out_ref, n):
        for o in range(0, (n // B) * B, B):
            pltpu.async_copy(accum.at[pl.ds(rbase + o, B)],
                             out_ref.at[pl.ds(rbase + o, B)], sem)
        tail = n - (n // B) * B
        if tail:
            pltpu.async_copy(accum.at[pl.ds(rbase + (n // B) * B, tail)],
                             out_ref.at[pl.ds(rbase + (n // B) * B, tail)],
                             sem)
        for o in range(0, (n // B) * B, B):
            pltpu.make_async_copy(accum.at[pl.ds(rbase, B)],
                                  out_ref.at[pl.ds(rbase, B)], sem).wait()
        if tail:
            pltpu.make_async_copy(accum.at[pl.ds(rbase, tail)],
                                  out_ref.at[pl.ds(rbase, tail)], sem).wait()

    for c, out_ref in ((0, out0_ref), (1, out1_ref)):
        @pl.when((cid == c) & (sid < NS - 1))
        def _(out_ref=out_ref):
            _writeout(out_ref, RPT)

        @pl.when((cid == c) & (sid == NS - 1))
        def _(out_ref=out_ref):
            _writeout(out_ref, RPT_LAST)


def kernel(input, edge_index, edge_weight, W):
    support = _tc_support(input, W)
    row = edge_index[0].astype(jnp.int32)
    col = edge_index[1].astype(jnp.int32)
    o0, o1 = _sc_spmm(support, row, col, edge_weight.astype(jnp.float32))
    return _tc_add(o0, o1)


# speedup vs baseline: 2.8968x; 1.0879x over previous
import functools

import jax
import jax.numpy as jnp
from jax import lax
from jax.experimental import pallas as pl
from jax.experimental.pallas import tpu as pltpu
from jax.experimental.pallas import tpu_sc as plsc

N_NODES = 10000
N_EDGES = 320000
F = 128
L = 16
NC = 2
NS = 16
NW = NC * NS
B = 104
WPAD = ((B - 1) // L) * L + L
EPT = N_EDGES // NW
NFULL = EPT // B
REM = EPT - NFULL * B
RPT = 624
RPT_LAST = N_NODES - (NS - 1) * RPT
BJORCK_ITER = 10
BJORCK_BETA = 0.5


def _tc_body(x_ref, w_ref, out_ref):
    prec = lax.Precision.HIGHEST

    def mm(a, b, dims):
        return lax.dot_general(a, b, (dims, ((), ())), precision=prec,
                               preferred_element_type=jnp.float32)

    v = w_ref[...] * (1.0 / 128.0)

    def it(_, v):
        vvt = mm(v, v, ((1,), (1,)))
        return (1.0 + BJORCK_BETA) * v - BJORCK_BETA * mm(vvt, v, ((1,), (0,)))

    v = lax.fori_loop(0, BJORCK_ITER, it, v)
    out_ref[...] = mm(x_ref[...], v, ((1,), (0,)))


_tc_support = pl.pallas_call(
    _tc_body,
    out_shape=jax.ShapeDtypeStruct((N_NODES, F), jnp.float32),
)


def _tc_add_body(a_ref, b_ref, out_ref):
    out_ref[...] = a_ref[...] + b_ref[...]


_tc_add = pl.pallas_call(
    _tc_add_body,
    out_shape=jax.ShapeDtypeStruct((N_NODES, F), jnp.float32),
)


_sc_mesh = plsc.VectorSubcoreMesh(core_axis_name="c", subcore_axis_name="s")

_UNROLL = 6
_NSUPER = NFULL // _UNROLL
assert _NSUPER * _UNROLL == NFULL

_DNUMS = lax.GatherDimensionNumbers(
    offset_dims=(), collapsed_slice_dims=(0,), start_index_map=(0,))


@functools.partial(
    pl.kernel,
    out_type=(jax.ShapeDtypeStruct((N_NODES, F), jnp.float32),
              jax.ShapeDtypeStruct((N_NODES, F), jnp.float32)),
    mesh=_sc_mesh,
    scratch_types=[
        pltpu.VMEM((6, B), jnp.int32),
        pltpu.VMEM((6, B), jnp.int32),
        pltpu.VMEM((6, WPAD), jnp.float32),
        pltpu.VMEM((3, B, F), jnp.float32),
        pltpu.VMEM((REM,), jnp.int32),
        pltpu.VMEM((REM,), jnp.int32),
        pltpu.VMEM((REM,), jnp.float32),
        pltpu.VMEM((REM, F), jnp.float32),
        pltpu.VMEM_SHARED((N_NODES, F), jnp.float32),
        pltpu.SemaphoreType.DMA,
        pltpu.SemaphoreType.DMA,
        pltpu.SemaphoreType.DMA,
        pltpu.SemaphoreType.DMA,
        pltpu.SemaphoreType.DMA,
        pltpu.SemaphoreType.DMA,
        pltpu.SemaphoreType.DMA,
        pltpu.SemaphoreType.DMA,
        pltpu.SemaphoreType.DMA,
        pltpu.SemaphoreType.DMA,
        pltpu.SemaphoreType.DMA,
        pltpu.SemaphoreType.DMA,
        pltpu.SemaphoreType.DMA,
    ],
)
def _sc_spmm(sup_ref, row_ref, col_ref, wt_ref, out0_ref, out1_ref,
             rowi_v, coli_v, wtc_v, rows_v,
             rowr_v, colr_v, wtr_v, rowsr_v,
             accum, asem0, asem1, asem2, asem3, asem4, asem5,
             gsem0, gsem1, gsem2, ssem0, ssem1, ssem2, sem):
    cid = lax.axis_index("c")
    sid = lax.axis_index("s")
    rbase = sid * RPT
    ebase = (cid * NS + sid) * EPT
    asems = (asem0, asem1, asem2, asem3, asem4, asem5)
    gsems = (gsem0, gsem1, gsem2)
    ssems = (ssem0, ssem1, ssem2)

    def _issue_a(i6, k):
        off = ebase + k * B
        pltpu.async_copy(row_ref.at[pl.ds(off, B)], rowi_v.at[i6], asems[i6])
        pltpu.async_copy(col_ref.at[pl.ds(off, B)], coli_v.at[i6], asems[i6])
        pltpu.async_copy(wt_ref.at[pl.ds(off, B)],
                         wtc_v.at[i6, pl.ds(0, B)], asems[i6])

    def _wait_a(i6):
        pltpu.make_async_copy(row_ref.at[pl.ds(0, B)], rowi_v.at[i6],
                              asems[i6]).wait()
        pltpu.make_async_copy(col_ref.at[pl.ds(0, B)], coli_v.at[i6],
                              asems[i6]).wait()
        pltpu.make_async_copy(wt_ref.at[pl.ds(0, B)],
                              wtc_v.at[i6, pl.ds(0, B)], asems[i6]).wait()

    def _issue_b(r3, i6):
        pltpu.async_copy(sup_ref.at[coli_v.at[i6]], rows_v.at[r3], gsems[r3])

    def _wait_b(r3, i6):
        pltpu.make_async_copy(sup_ref.at[coli_v.at[i6]], rows_v.at[r3],
                              gsems[r3]).wait()

    def _issue_s(r3, i6):
        pltpu.async_copy(rows_v.at[r3], accum.at[rowi_v.at[i6]], ssems[r3],
                         add=True)

    def _wait_s(r3, i6):
        pltpu.make_async_copy(rows_v.at[r3], accum.at[rowi_v.at[i6]],
                              ssems[r3]).wait()

    def _multiply(rows, wtc, n):
        @plsc.parallel_loop(0, n, 1, unroll=4)
        def _(e):
            g16 = (e // L) * L
            w = wtc[pl.ds(g16, L)]
            w16 = lax.gather(
                w, lax.broadcast(e - g16, (L,))[:, None], _DNUMS,
                slice_sizes=(1,),
                mode=lax.GatherScatterMode.PROMISE_IN_BOUNDS)
            for j in range(F // L):
                sl = pl.ds(j * L, L)
                rows[e, sl] = rows[e, sl] * w16

    def _zero_rows(e, c):
        for j in range(F // L):
            rowsr_v[e, pl.ds(j * L, L)] = jnp.zeros((L,), jnp.float32)
        return c

    lax.fori_loop(0, REM, _zero_rows, 0)

    def _zero_range(start, n):
        for o in range(0, (n // REM) * REM, REM):
            pltpu.async_copy(rowsr_v, accum.at[pl.ds(start + o, REM)], sem)

    def _zero_drain(n):
        for o in range(0, (n // REM) * REM, REM):
            pltpu.make_async_copy(rowsr_v, accum.at[pl.ds(rbase, REM)],
                                  sem).wait()

    @pl.when(sid < NS - 1)
    def _():
        _zero_range(rbase, RPT)

    @pl.when(sid == NS - 1)
    def _():
        _zero_range(rbase, RPT_LAST)

    _issue_a(0, 0)
    _issue_a(1, 1)
    _issue_a(2, 2)
    _issue_a(3, 3)

    @pl.when(sid < NS - 1)
    def _():
        _zero_drain(RPT)

    @pl.when(sid == NS - 1)
    def _():
        _zero_drain(RPT_LAST)

    plsc.subcore_barrier()

    _wait_a(0)
    _issue_b(0, 0)
    _wait_a(1)
    _issue_b(1, 1)

    def _super(sup, c):
        k0 = sup * _UNROLL
        for j in range(_UNROLL):
            kk = k0 + j
            r3 = j % 3
            _wait_b(r3, j)

            @pl.when(kk >= 1)
            def _():
                _wait_s((j + 2) % 3, (j + 5) % 6)

            @pl.when(kk + 2 < NFULL)
            def _():
                _wait_a((j + 2) % 6)
                _issue_b((j + 2) % 3, (j + 2) % 6)

            @pl.when(kk + 4 < NFULL)
            def _():
                _issue_a((j + 4) % 6, kk + 4)

            _multiply(rows_v.at[r3], wtc_v.at[j], B)
            _issue_s(r3, j)
        return c

    lax.fori_loop(0, _NSUPER, _super, 0)
    _wait_s((NFULL - 1) % 3, (NFULL - 1) % 6)

    if REM:
        off = ebase + NFULL * B
        pltpu.sync_copy(row_ref.at[pl.ds(off, REM)], rowr_v)
        pltpu.sync_copy(col_ref.at[pl.ds(off, REM)], colr_v)
        pltpu.sync_copy(wt_ref.at[pl.ds(off, REM)], wtr_v)
        pltpu.async_copy(sup_ref.at[colr_v], rowsr_v, sem).wait()
        _multiply(rowsr_v, wtr_v, REM)
        pltpu.sync_copy(rowsr_v, accum.at[rowr_v], add=True)
    plsc.subcore_barrier()

    def _writeout(out_ref, n):
        for o in range(0, (n // B) * B, B):
            pltpu.async_copy(accum.at[pl.ds(rbase + o, B)],
                             out_ref.at[pl.ds(rbase + o, B)], sem)
        tail = n - (n // B) * B
        if tail:
            pltpu.async_copy(accum.at[pl.ds(rbase + (n // B) * B, tail)],
                             out_ref.at[pl.ds(rbase + (n // B) * B, tail)],
                             sem)
        for o in range(0, (n // B) * B, B):
            pltpu.make_async_copy(accum.at[pl.ds(rbase, B)],
                                  out_ref.at[pl.ds(rbase, B)], sem).wait()
        if tail:
            pltpu.make_async_copy(accum.at[pl.ds(rbase, tail)],
                                  out_ref.at[pl.ds(rbase, tail)], sem).wait()

    for c, out_ref in ((0, out0_ref), (1, out1_ref)):
        @pl.when((cid == c) & (sid < NS - 1))
        def _(out_ref=out_ref):
            _writeout(out_ref, RPT)

        @pl.when((cid == c) & (sid == NS - 1))
        def _(out_ref=out_ref):
            _writeout(out_ref, RPT_LAST)


def kernel(input, edge_index, edge_weight, W):
    support = _tc_support(input, W)
    row = edge_index[0].astype(jnp.int32)
    col = edge_index[1].astype(jnp.int32)
    o0, o1 = _sc_spmm(support, row, col, edge_weight.astype(jnp.float32))
    return _tc_add(o0, o1)
